# baseline (speedup 1.0000x reference)
import functools

import jax
import jax.numpy as jnp
from jax import lax
from jax.experimental import pallas as pl
from jax.experimental.pallas import tpu as pltpu
from jax.experimental.pallas import tpu_sc as plsc

N = 10000
E = 320000
D = 128
DH = D // 2
DW = 16
AW = DH + DW

NC = 2
NS = 16
CHUNK = 128
G = 2
GLEN = G * CHUNK
EPW = 20480
NG = EPW // GLEN
E_PAD = NS * EPW
N_PAD = 10240

SHIFT = 14
MASK = (1 << SHIFT) - 1

ZROWS = N_PAD // NS
ZHALF = ZROWS // 2
OROWS = 624
OTAIL = N - NS * OROWS


def _sc_aggregate(xal, xar, epacked):
    mesh = plsc.VectorSubcoreMesh(core_axis_name="c", subcore_axis_name="s")

    @functools.partial(
        pl.kernel, mesh=mesh,
        out_type=(
            jax.ShapeDtypeStruct((N, AW), jnp.float32),
            jax.ShapeDtypeStruct((N, AW), jnp.float32),
        ),
        scratch_types=[
            pltpu.VMEM_SHARED((N_PAD, AW), jnp.float32),
            pltpu.VMEM((2, GLEN), jnp.int32),
            pltpu.VMEM((2, G, CHUNK), jnp.int32),
            pltpu.VMEM((2, G, CHUNK), jnp.int32),
            pltpu.VMEM((2, G, CHUNK, AW), jnp.float32),
            pltpu.VMEM((ZHALF, AW), jnp.float32),
            pltpu.SemaphoreType.DMA,
            pltpu.SemaphoreType.DMA,
            pltpu.SemaphoreType.DMA,
            pltpu.SemaphoreType.DMA,
            pltpu.SemaphoreType.DMA,
            pltpu.SemaphoreType.DMA,
        ],
        compiler_params=pltpu.CompilerParams(use_tc_tiling_on_sc=False),
    )
    def k(xal_hbm, xar_hbm, e_hbm, aggl_out, aggr_out,
          agg_sp, pck_v, src_v, dst_v, rows_v, zagg_v,
          si0, si1, sg0, sg1, sa0, sa1):
        si = (si0, si1)
        sg = (sg0, sg1)
        sa = (sa0, sa1)

        c = lax.axis_index("c")
        s = lax.axis_index("s")
        on_sc0 = c == 0

        zero16 = jnp.zeros((16,), jnp.float32)

        @pl.loop(0, ZHALF)
        def _(i):
            @pl.loop(0, AW // 16)
            def _(j):
                zagg_v[i, pl.ds(j * 16, 16)] = zero16

        zrow = s * ZROWS
        pltpu.sync_copy(zagg_v, agg_sp.at[pl.ds(zrow, ZHALF)])
        pltpu.sync_copy(zagg_v, agg_sp.at[pl.ds(zrow + ZHALF, ZHALF)])
        plsc.subcore_barrier()

        grow = s * (EPW // GLEN)

        def issue_idx(g, b):
            pltpu.async_copy(e_hbm.at[grow + g], pck_v.at[b], si[b])

        def wait_idx(b):
            pltpu.make_async_copy(e_hbm.at[0], pck_v.at[b], si[b]).wait()

        def unpack(b):
            for j in range(G):
                @pl.loop(0, CHUNK // 16)
                def _(u):
                    p = pck_v[b, pl.ds(j * CHUNK + u * 16, 16)]
                    src_v[b, j, pl.ds(u * 16, 16)] = (
                        lax.shift_right_logical(p, SHIFT))
                    dst_v[b, j, pl.ds(u * 16, 16)] = lax.bitwise_and(p, MASK)

        def issue_gathers(b):
            @pl.when(on_sc0)
            def _():
                @pl.loop(0, G)
                def _(j):
                    pltpu.async_copy(xal_hbm.at[src_v.at[b, j]],
                                     rows_v.at[b, j], sg[b])

            @pl.when(~on_sc0)
            def _():
                @pl.loop(0, G)
                def _(j):
                    pltpu.async_copy(xar_hbm.at[src_v.at[b, j]],
                                     rows_v.at[b, j], sg[b])

        def wait_gathers(b):
            @pl.loop(0, G)
            def _(j):
                pltpu.make_async_copy(xal_hbm.at[src_v.at[b, j]],
                                      rows_v.at[b, j], sg[b]).wait()

        def issue_scatters(b):
            @pl.loop(0, G)
            def _(j):
                pltpu.async_copy(rows_v.at[b, j], agg_sp.at[dst_v.at[b, j]],
                                 sa[b], add=True)

        def wait_scatters(b):
            @pl.loop(0, G)
            def _(j):
                pltpu.make_async_copy(rows_v.at[b, j],
                                      agg_sp.at[dst_v.at[b, j]], sa[b]).wait()

        pltpu.sync_copy(e_hbm.at[grow], pck_v.at[0])
        unpack(0)
        issue_gathers(0)
        issue_idx(1, 1)

        @pl.loop(0, NG // 2)
        def _(t):
            for b in (0, 1):
                nxt = 1 - b

                def prep():
                    wait_idx(nxt)

                    def drain():
                        wait_scatters(nxt)
                    if b == 0:
                        pl.when(t > 0)(drain)
                    else:
                        drain()
                    unpack(nxt)

                    def fetch():
                        issue_idx(2 * t + b + 2, b)
                    if b == 0:
                        pl.when(t < NG // 2 - 1)(fetch)
                    else:
                        fetch()
                    issue_gathers(nxt)
                if b == 0:
                    prep()
                else:
                    pl.when(t < NG // 2 - 1)(prep)
                wait_gathers(b)
                issue_scatters(b)

        wait_scatters(0)
        wait_scatters(1)
        plsc.subcore_barrier()

        orow = s * OROWS

        def writeback(agg_dst):
            pltpu.sync_copy(agg_sp.at[pl.ds(orow, OROWS)],
                            agg_dst.at[pl.ds(orow, OROWS)])

            @pl.when(s == NS - 1)
            def _():
                tail = NS * OROWS
                pltpu.sync_copy(agg_sp.at[pl.ds(tail, OTAIL)],
                                agg_dst.at[pl.ds(tail, OTAIL)])

        @pl.when(on_sc0)
        def _():
            writeback(aggl_out)

        @pl.when(~on_sc0)
        def _():
            writeback(aggr_out)

    return k(xal, xar, epacked)


BLK = 1000


def _tc_body(x_ref, al_ref, ar_ref, w_ref, b_ref, o_ref):
    inv = 1.0 / jnp.maximum(al_ref[:, DH:DH + 1], 1.0)
    h = jnp.dot(x_ref[...], w_ref[0:D, :], preferred_element_type=jnp.float32)
    h += jnp.dot(al_ref[:, :DH] * inv, w_ref[D:D + DH, :],
                 preferred_element_type=jnp.float32)
    h += jnp.dot(ar_ref[:, :DH] * inv, w_ref[D + DH:2 * D, :],
                 preferred_element_type=jnp.float32)
    o_ref[...] = jnp.maximum(h + b_ref[...], 0.0)


def _tc_linear(x, aggl, aggr, w, b2):
    return pl.pallas_call(
        _tc_body,
        grid=(N // BLK,),
        in_specs=[
            pl.BlockSpec((BLK, D), lambda i: (i, 0)),
            pl.BlockSpec((BLK, AW), lambda i: (i, 0)),
            pl.BlockSpec((BLK, AW), lambda i: (i, 0)),
            pl.BlockSpec((2 * D, D), lambda i: (0, 0)),
            pl.BlockSpec((1, D), lambda i: (0, 0)),
        ],
        out_specs=pl.BlockSpec((BLK, D), lambda i: (i, 0)),
        out_shape=jax.ShapeDtypeStruct((N, D), jnp.float32),
    )(x, aggl, aggr, w, b2)


def kernel(X, edge_index, W, b):
    src = edge_index[0].astype(jnp.int32)
    dst = edge_index[1].astype(jnp.int32)
    pad = E_PAD - E
    packed = jnp.concatenate([
        jnp.left_shift(src, SHIFT) | dst,
        jnp.full((pad,), N, jnp.int32),
    ])
    ones = jnp.ones((N, DW), jnp.float32)
    xal = jnp.concatenate([X[:, :DH], ones], axis=1)
    xar = jnp.concatenate([X[:, DH:], ones], axis=1)
    aggl, aggr = _sc_aggregate(xal, xar, packed.reshape(E_PAD // GLEN, GLEN))
    return _tc_linear(X, aggl, aggr, W, b.reshape(1, D))
